# baseline (speedup 1.0000x reference)
import functools

import jax
import jax.numpy as jnp
from jax import lax
from jax.experimental import pallas as pl
from jax.experimental.pallas import tpu as pltpu
from jax.experimental.pallas import tpu_sc as plsc

F = 64
B = 16384
SEG = 128
N_ROWS = 1000000
R = 8192
NB = 30
Q = NB * R
TAIL = 4 * Q
OUT_ROWS = Q + (N_ROWS - TAIL)
TS = 512


def _fuse_tables_tc(ut, vt):
    def pack_pair(t):
        bits = lax.bitcast_convert_type(t, jnp.int32)
        rnd = bits + 0x7FFF + jnp.bitwise_and(
            lax.shift_right_logical(bits, 16), 1)
        lo = lax.shift_right_logical(rnd[:, 0:32], 16)
        hi = jnp.bitwise_and(rnd[:, 32:64], -65536)
        return jnp.bitwise_or(lo, hi)

    def body(u0, u1, u2, u3, v0, v1, v2, v3, u_out, v_out):
        for srcs, dst in (((u0, u1, u2, u3), u_out),
                          ((v0, v1, v2, v3), v_out)):
            for q in range(4):
                for k in range(R // TS):
                    sl = pl.ds(k * TS, TS)
                    dst[sl, pl.ds(32 * q, 32)] = pack_pair(srcs[q][:, sl].T)

    def qmap(q):
        def m(i):
            return (0, jnp.where(i < NB, i + q * NB, 4 * NB))
        return m

    qspecs = [pl.BlockSpec((F, R), qmap(q)) for q in range(4)]
    out_spec = pl.BlockSpec((R, 2 * F), lambda i: (i, 0))
    out_shape = jax.ShapeDtypeStruct((OUT_ROWS, 2 * F), jnp.int32)
    return pl.pallas_call(
        body,
        grid=(NB + 1,),
        in_specs=qspecs + qspecs,
        out_specs=[out_spec, out_spec],
        out_shape=[out_shape, out_shape],
    )(ut, ut, ut, ut, vt, vt, vt, vt)


def _gmf_sc(x1r, x2r, u_fused, v_fused, wb):
    info = plsc.get_sparse_core_info()
    nw = info.num_cores * info.num_subcores
    b_per_w = B // nw
    n_seg = b_per_w // SEG

    mesh = plsc.VectorSubcoreMesh(core_axis_name="c", subcore_axis_name="s")

    @functools.partial(
        pl.kernel,
        mesh=mesh,
        out_type=jax.ShapeDtypeStruct((B,), jnp.float32),
        scratch_types=[
            pltpu.VMEM((n_seg, SEG), jnp.int32),
            pltpu.VMEM((n_seg, SEG), jnp.int32),
            pltpu.VMEM((n_seg, SEG), jnp.int32),
            pltpu.VMEM((n_seg, SEG), jnp.int32),
            pltpu.VMEM((b_per_w,), jnp.int32),
            pltpu.VMEM((b_per_w,), jnp.int32),
            pltpu.VMEM((2, SEG, 2 * F), jnp.int32),
            pltpu.VMEM((2, SEG, 2 * F), jnp.int32),
            pltpu.VMEM((b_per_w,), jnp.float32),
            pltpu.VMEM((F + 16,), jnp.float32),
            pltpu.SemaphoreType.DMA,
            pltpu.SemaphoreType.DMA,
        ],
    )
    def k(x1_hbm, x2_hbm, u_hbm, v_hbm, wb_hbm, out_hbm,
          idx1_v, idx2_v, g1_v, g2_v, p1_v, p2_v, u_v, v_v, out_v, wb_v,
          sem0, sem1):
        wid = lax.axis_index("s") * info.num_cores + lax.axis_index("c")
        base = wid * b_per_w

        pltpu.sync_copy(x1_hbm.at[pl.ds(wid * n_seg, n_seg)], idx1_v)
        pltpu.sync_copy(x2_hbm.at[pl.ds(wid * n_seg, n_seg)], idx2_v)
        pltpu.sync_copy(wb_hbm, wb_v)

        for j in range(n_seg):
            for c in range(SEG // 16):
                sl = pl.ds(16 * c, 16)
                fl = pl.ds(j * SEG + 16 * c, 16)
                for raw, g_v, p_v in ((idx1_v[j, sl], g1_v, p1_v),
                                      (idx2_v[j, sl], g2_v, p2_v)):
                    one = jnp.ones((16,), jnp.int32)
                    zero = jnp.zeros((16,), jnp.int32)
                    qtr = (jnp.where(raw >= Q, one, zero)
                           + jnp.where(raw >= 2 * Q, one, zero)
                           + jnp.where(raw >= 3 * Q, one, zero))
                    g_v[j, sl] = raw - qtr * Q
                    p_v[fl] = jnp.where(raw >= TAIL, 0, qtr * 32)

        sems = (sem0, sem1)

        def fire(j):
            buf = j % 2
            return (
                pltpu.async_copy(u_hbm.at[g1_v.at[j]], u_v.at[buf],
                                 sems[buf]),
                pltpu.async_copy(v_hbm.at[g2_v.at[j]], v_v.at[buf],
                                 sems[buf]),
            )

        w0 = wb_v[pl.ds(0, 16)]
        w1 = wb_v[pl.ds(16, 16)]
        w2 = wb_v[pl.ds(32, 16)]
        w3 = wb_v[pl.ds(48, 16)]
        bias = wb_v[pl.ds(F, 16)]
        lane = lax.iota(jnp.int32, 16)
        perms = [jnp.bitwise_xor(lane, sh) for sh in (8, 4, 2, 1)]

        def lanesum(v):
            for p in perms:
                v = v + v.at[p].get(mode="promise_in_bounds")
            return v

        def unpack2(w):
            lo = lax.bitcast_convert_type(lax.shift_left(w, 16), jnp.float32)
            hi = lax.bitcast_convert_type(jnp.bitwise_and(w, -65536),
                                          jnp.float32)
            return lo, hi

        def make_group_body(j):
            buf = j % 2
            ub = u_v.at[buf]
            vb = v_v.at[buf]

            def group_body(g, carry):
                base_r = pl.multiple_of(g * 16, 16)
                pv1 = p1_v[pl.ds(j * SEG + base_r, 16)]
                pv2 = p2_v[pl.ds(j * SEG + base_r, 16)]
                acc = jnp.zeros((16,), jnp.float32)
                for r in range(16):
                    i = base_r + r
                    o1 = pl.multiple_of(pv1[r], 32)
                    o2 = pl.multiple_of(pv2[r], 32)
                    u0, u2 = unpack2(ub[i, pl.ds(o1, 16)])
                    u1, u3 = unpack2(ub[i, pl.ds(o1 + 16, 16)])
                    v0, v2 = unpack2(vb[i, pl.ds(o2, 16)])
                    v1, v3 = unpack2(vb[i, pl.ds(o2 + 16, 16)])
                    s = (u0 * v0) * w0
                    s = s + (u1 * v1) * w1
                    s = s + (u2 * v2) * w2
                    s = s + (u3 * v3) * w3
                    acc = jnp.where(lane == r, lanesum(s), acc)
                x = acc + bias
                out_v[pl.ds(j * SEG + base_r, 16)] = 1.0 / (1.0 + jnp.exp(-x))
                return carry

            return group_body

        inflight = fire(0)
        for j in range(n_seg):
            nxt = fire(j + 1) if j + 1 < n_seg else None
            for c in inflight:
                c.wait()
            lax.fori_loop(0, SEG // 16, make_group_body(j), 0)
            inflight = nxt

        pltpu.sync_copy(out_v, out_hbm.at[pl.ds(base, b_per_w)])

    return k(x1r, x2r, u_fused, v_fused, wb)


def kernel(x1, x2, user_table, item_table, W, b):
    x1r = x1.reshape(B // SEG, SEG)
    x2r = x2.reshape(B // SEG, SEG)
    wb = jnp.concatenate([W.reshape(F), jnp.broadcast_to(b, (16,))])
    u_fused, v_fused = _fuse_tables_tc(user_table.T, item_table.T)
    out = _gmf_sc(x1r, x2r, u_fused, v_fused, wb)
    return out.reshape(B, 1)


# speedup vs baseline: 1.6927x; 1.6927x over previous
import functools

import jax
import jax.numpy as jnp
from jax import lax
from jax.experimental import pallas as pl
from jax.experimental.pallas import tpu as pltpu
from jax.experimental.pallas import tpu_sc as plsc

F = 64
B = 16384
SEG = 128
N_ROWS = 1000000
R = 8192
NB = 61
HALF = NB * R
TAIL = 2 * HALF
OUT_ROWS = HALF + (N_ROWS - TAIL)
TAIL_BLOCKS = -(-(N_ROWS - TAIL) // R)
TS = 512


def _fuse_tables_tc(ut, vt):
    def body(u_top, u_bot, v_top, v_bot, u_out, v_out):
        for src, dst, col in ((u_top, u_out, 0), (u_bot, u_out, F),
                              (v_top, v_out, 0), (v_bot, v_out, F)):
            for k in range(R // TS):
                sl = pl.ds(k * TS, TS)
                dst[sl, pl.ds(col, F)] = src[:, sl].T

    def top_map(i):
        return (0, jnp.where(i < NB, i, i + NB))

    def bot_map(i):
        return (0, i + NB)

    top_spec = pl.BlockSpec((F, R), top_map)
    bot_spec = pl.BlockSpec((F, R), bot_map)
    out_spec = pl.BlockSpec((R, 2 * F), lambda i: (i, 0))
    out_shape = jax.ShapeDtypeStruct((OUT_ROWS, 2 * F), jnp.float32)
    return pl.pallas_call(
        body,
        grid=(NB + TAIL_BLOCKS,),
        in_specs=[top_spec, bot_spec, top_spec, bot_spec],
        out_specs=[out_spec, out_spec],
        out_shape=[out_shape, out_shape],
    )(ut, ut, vt, vt)


def _gmf_sc(x1r, x2r, u_fused, v_fused, wb):
    info = plsc.get_sparse_core_info()
    nw = info.num_cores * info.num_subcores
    b_per_w = B // nw
    n_seg = b_per_w // SEG

    mesh = plsc.VectorSubcoreMesh(core_axis_name="c", subcore_axis_name="s")

    @functools.partial(
        pl.kernel,
        mesh=mesh,
        out_type=jax.ShapeDtypeStruct((B,), jnp.float32),
        scratch_types=[
            pltpu.VMEM((n_seg, SEG), jnp.int32),
            pltpu.VMEM((n_seg, SEG), jnp.int32),
            pltpu.VMEM((n_seg, SEG), jnp.int32),
            pltpu.VMEM((n_seg, SEG), jnp.int32),
            pltpu.VMEM((b_per_w,), jnp.int32),
            pltpu.VMEM((b_per_w,), jnp.int32),
            pltpu.VMEM((2, SEG, 2 * F), jnp.float32),
            pltpu.VMEM((2, SEG, 2 * F), jnp.float32),
            pltpu.VMEM((b_per_w,), jnp.float32),
            pltpu.VMEM((F + 16,), jnp.float32),
            pltpu.SemaphoreType.DMA,
            pltpu.SemaphoreType.DMA,
        ],
    )
    def k(x1_hbm, x2_hbm, u_hbm, v_hbm, wb_hbm, out_hbm,
          idx1_v, idx2_v, g1_v, g2_v, p1_v, p2_v, u_v, v_v, out_v, wb_v,
          sem0, sem1):
        wid = lax.axis_index("s") * info.num_cores + lax.axis_index("c")
        base = wid * b_per_w

        pltpu.sync_copy(x1_hbm.at[pl.ds(wid * n_seg, n_seg)], idx1_v)
        pltpu.sync_copy(x2_hbm.at[pl.ds(wid * n_seg, n_seg)], idx2_v)
        pltpu.sync_copy(wb_hbm, wb_v)

        for j in range(n_seg):
            for c in range(SEG // 16):
                sl = pl.ds(16 * c, 16)
                raw1 = idx1_v[j, sl]
                raw2 = idx2_v[j, sl]
                hi1 = raw1 >= HALF
                hi2 = raw2 >= HALF
                g1_v[j, sl] = raw1 - jnp.where(hi1, HALF, 0)
                g2_v[j, sl] = raw2 - jnp.where(hi2, HALF, 0)
                fl = pl.ds(j * SEG + 16 * c, 16)
                p1_v[fl] = jnp.where(hi1 & (raw1 < TAIL), 64, 0)
                p2_v[fl] = jnp.where(hi2 & (raw2 < TAIL), 64, 0)

        sems = (sem0, sem1)

        def fire(j):
            buf = j % 2
            return (
                pltpu.async_copy(u_hbm.at[g1_v.at[j]], u_v.at[buf],
                                 sems[buf]),
                pltpu.async_copy(v_hbm.at[g2_v.at[j]], v_v.at[buf],
                                 sems[buf]),
            )

        w0 = wb_v[pl.ds(0, 16)]
        w1 = wb_v[pl.ds(16, 16)]
        w2 = wb_v[pl.ds(32, 16)]
        w3 = wb_v[pl.ds(48, 16)]
        bias = wb_v[pl.ds(F, 16)]
        lane = lax.iota(jnp.int32, 16)
        perms = [jnp.bitwise_xor(lane, sh) for sh in (8, 4, 2, 1)]

        def lanesum(v):
            for p in perms:
                v = v + v.at[p].get(mode="promise_in_bounds")
            return v

        def make_group_body(j):
            buf = j % 2
            ub = u_v.at[buf]
            vb = v_v.at[buf]

            def group_body(g, carry):
                base_r = pl.multiple_of(g * 16, 16)
                pv1 = p1_v[pl.ds(j * SEG + base_r, 16)]
                pv2 = p2_v[pl.ds(j * SEG + base_r, 16)]
                acc = jnp.zeros((16,), jnp.float32)
                for r in range(16):
                    i = base_r + r
                    o1 = pl.multiple_of(pv1[r], 16)
                    o2 = pl.multiple_of(pv2[r], 16)
                    s = (ub[i, pl.ds(o1, 16)] * vb[i, pl.ds(o2, 16)]) * w0
                    s = s + (ub[i, pl.ds(o1 + 16, 16)]
                             * vb[i, pl.ds(o2 + 16, 16)]) * w1
                    s = s + (ub[i, pl.ds(o1 + 32, 16)]
                             * vb[i, pl.ds(o2 + 32, 16)]) * w2
                    s = s + (ub[i, pl.ds(o1 + 48, 16)]
                             * vb[i, pl.ds(o2 + 48, 16)]) * w3
                    acc = jnp.where(lane == r, lanesum(s), acc)
                x = acc + bias
                out_v[pl.ds(j * SEG + base_r, 16)] = 1.0 / (1.0 + jnp.exp(-x))
                return carry

            return group_body

        inflight = fire(0)
        for j in range(n_seg):
            nxt = fire(j + 1) if j + 1 < n_seg else None
            for c in inflight:
                c.wait()
            lax.fori_loop(0, SEG // 16, make_group_body(j), 0)
            inflight = nxt

        pltpu.sync_copy(out_v, out_hbm.at[pl.ds(base, b_per_w)])

    return k(x1r, x2r, u_fused, v_fused, wb)


def kernel(x1, x2, user_table, item_table, W, b):
    x1r = x1.reshape(B // SEG, SEG)
    x2r = x2.reshape(B // SEG, SEG)
    wb = jnp.concatenate([W.reshape(F), jnp.broadcast_to(b, (16,))])
    u_fused, v_fused = _fuse_tables_tc(user_table.T, item_table.T)
    out = _gmf_sc(x1r, x2r, u_fused, v_fused, wb)
    return out.reshape(B, 1)
